# baseline (speedup 1.0000x reference)
import jax
import jax.numpy as jnp
from jax import lax
from jax.experimental import pallas as pl
from jax.experimental.pallas import tpu as pltpu
from jax.experimental.pallas import tpu_sc as plsc


_TC_GRID = 32


def _tc_copy_body(k_ref, ok_ref):
    ok_ref[...] = k_ref[...]


def _tc_copy(x):
    n, hd = x.shape
    rows = n // _TC_GRID
    x3 = x.reshape(_TC_GRID, rows, hd)
    spec = pl.BlockSpec((1, rows, hd), lambda i: (i, 0, 0))
    out = pl.pallas_call(
        _tc_copy_body,
        grid=(_TC_GRID,),
        out_shape=jax.ShapeDtypeStruct(x3.shape, x3.dtype),
        in_specs=[spec],
        out_specs=spec,
        compiler_params=pltpu.CompilerParams(
            dimension_semantics=("parallel",),
        ),
    )(x3)
    return out.reshape(n, hd)


_NC = 2
_NS = 16
_NW = _NC * _NS
_CH = 256


def _sc_copy_body(v_hbm, out_hbm, buf0, buf1, si0, si1, so0, so1):
    rows_per_w = v_hbm.shape[0] // _NW
    nch = rows_per_w // _CH
    wid = lax.axis_index("s") * _NC + lax.axis_index("c")
    base = wid * rows_per_w
    bufs = (buf0, buf1)
    sin = (si0, si1)
    sout = (so0, so1)

    def make_in(c):
        b = c & 1
        return pltpu.make_async_copy(
            v_hbm.at[pl.ds(base + c * _CH, _CH)], bufs[b], sin[b]
        )

    def make_out(c):
        b = c & 1
        return pltpu.make_async_copy(
            bufs[b], out_hbm.at[pl.ds(base + c * _CH, _CH)], sout[b]
        )

    make_in(0).start()
    for c in range(nch):
        make_in(c).wait()
        make_out(c).start()
        if c + 1 < nch:
            if c >= 1:
                make_out(c - 1).wait()
            make_in(c + 1).start()
    if nch >= 2:
        make_out(nch - 2).wait()
    make_out(nch - 1).wait()


def _sc_copy(x):
    n, hd = x.shape
    mesh = plsc.VectorSubcoreMesh(core_axis_name="c", subcore_axis_name="s")
    f = pl.kernel(
        _sc_copy_body,
        out_type=jax.ShapeDtypeStruct((n, hd), x.dtype),
        mesh=mesh,
        scratch_types=[
            pltpu.VMEM((_CH, hd), x.dtype),
            pltpu.VMEM((_CH, hd), x.dtype),
            pltpu.SemaphoreType.DMA,
            pltpu.SemaphoreType.DMA,
            pltpu.SemaphoreType.DMA,
            pltpu.SemaphoreType.DMA,
        ],
    )
    return f(x)


def kernel(new_k, new_v):
    B, H, L, HD = new_k.shape
    k2 = new_k.reshape(B * H * L, HD)
    v2 = new_v.reshape(B * H * L, HD)
    ok = _tc_copy(k2)
    ov = _sc_copy(v2)
    return ok.reshape(B, H, L, HD), ov.reshape(B, H, L, HD)
